# Initial kernel scaffold; baseline (speedup 1.0000x reference)
import jax
import jax.numpy as jnp
from jax.experimental import pallas as pl


def kernel(src, categories_means, categories_logvars):
    raise NotImplementedError("write your pallas kernel here")


# speedup vs baseline: 1.1353x; 1.1353x over previous
import functools

import jax
import jax.numpy as jnp
from jax import lax
from jax.experimental import pallas as pl
from jax.experimental.pallas import tpu as pltpu
from jax.experimental.pallas import tpu_sc as plsc

BATCH = 16384
FIELDS = 26
OUT = 32
TOTAL = BATCH * FIELDS
NUM_CORES = 2
NUM_SUBCORES = 16
NW = NUM_CORES * NUM_SUBCORES
PER_W = TOTAL // NW
CHUNK = 1024
N_CHUNKS = PER_W // CHUNK
assert PER_W * NW == TOTAL and N_CHUNKS * CHUNK == PER_W


def _rsqrt(x):
    i = lax.bitcast_convert_type(x, jnp.int32)
    i = jnp.full((16,), 0x5F3759DF, jnp.int32) - (i >> 1)
    y = lax.bitcast_convert_type(i, jnp.float32)
    for _ in range(3):
        y = y * (1.5 - 0.5 * x * y * y)
    return y


_mesh = plsc.VectorSubcoreMesh(core_axis_name="c", subcore_axis_name="s")


@functools.partial(
    pl.kernel,
    out_type=jax.ShapeDtypeStruct((TOTAL, OUT), jnp.float32),
    mesh=_mesh,
    scratch_types=[
        pltpu.VMEM((CHUNK,), jnp.int32),
        pltpu.VMEM((CHUNK, OUT), jnp.float32),
        pltpu.SemaphoreType.DMA,
    ],
    compiler_params=pltpu.CompilerParams(
        needs_layout_passes=False, use_tc_tiling_on_sc=False
    ),
)
def _gather_normalize(table_hbm, idx_hbm, out_hbm, idx_v, rows_v, sem):
    wid = lax.axis_index("s") * NUM_CORES + lax.axis_index("c")
    base = wid * PER_W

    lanes = lax.iota(jnp.int32, 16)
    cols = [jnp.full((16,), j, jnp.int32) for j in range(OUT)]

    def chunk_body(ci, carry):
        cbase = base + ci * CHUNK
        pltpu.sync_copy(idx_hbm.at[pl.ds(cbase, CHUNK)], idx_v)
        pltpu.async_copy(table_hbm.at[idx_v], rows_v, sem).wait()

        def group_body(g, c):
            row_ids = g * 16 + lanes
            elems = [plsc.load_gather(rows_v, [row_ids, cols[j]])
                     for j in range(OUT)]
            acc = jnp.zeros((16,), jnp.float32)
            for e in elems:
                acc = acc + e * e
            inv = _rsqrt(acc)
            for j, e in enumerate(elems):
                plsc.store_scatter(rows_v, [row_ids, cols[j]], e * inv)
            return c

        lax.fori_loop(0, CHUNK // 16, group_body, 0)
        pltpu.sync_copy(rows_v, out_hbm.at[pl.ds(cbase, CHUNK)])
        return carry

    lax.fori_loop(0, N_CHUNKS, chunk_body, 0)


def kernel(src, categories_means, categories_logvars):
    del categories_logvars
    idx = src.reshape(TOTAL).astype(jnp.int32)
    flat = _gather_normalize(categories_means, idx)
    return flat.reshape(BATCH, FIELDS, OUT)


# speedup vs baseline: 1.1822x; 1.0413x over previous
import functools

import jax
import jax.numpy as jnp
from jax import lax
from jax.experimental import pallas as pl
from jax.experimental.pallas import tpu as pltpu
from jax.experimental.pallas import tpu_sc as plsc

BATCH = 16384
FIELDS = 26
OUT = 32
TOTAL = BATCH * FIELDS
NUM_CORES = 2
NUM_SUBCORES = 16
NW = NUM_CORES * NUM_SUBCORES
PER_W = TOTAL // NW
CHUNK = 512
N_CHUNKS = PER_W // CHUNK
GROUPS = CHUNK // 16
assert PER_W * NW == TOTAL and N_CHUNKS * CHUNK == PER_W
assert N_CHUNKS % 2 == 0


def _rsqrt(x):
    i = lax.bitcast_convert_type(x, jnp.int32)
    i = jnp.full((16,), 0x5F3759DF, jnp.int32) - (i >> 1)
    y = lax.bitcast_convert_type(i, jnp.float32)
    for _ in range(3):
        y = y * (1.5 - 0.5 * x * y * y)
    return y


_mesh = plsc.VectorSubcoreMesh(core_axis_name="c", subcore_axis_name="s")


@functools.partial(
    pl.kernel,
    out_type=jax.ShapeDtypeStruct((TOTAL, OUT), jnp.float32),
    mesh=_mesh,
    scratch_types=[
        pltpu.VMEM((N_CHUNKS, CHUNK), jnp.int32),
        pltpu.VMEM((CHUNK, OUT), jnp.float32),
        pltpu.VMEM((CHUNK, OUT), jnp.float32),
        pltpu.VMEM((CHUNK, OUT), jnp.float32),
        pltpu.VMEM((CHUNK, OUT), jnp.float32),
        pltpu.SemaphoreType.DMA,
        pltpu.SemaphoreType.DMA,
        pltpu.SemaphoreType.DMA,
        pltpu.SemaphoreType.DMA,
    ],
    compiler_params=pltpu.CompilerParams(
        needs_layout_passes=False, use_tc_tiling_on_sc=False
    ),
)
def _gather_normalize(table_hbm, idx_hbm, out_hbm,
                      idx_v, g0, g1, o0, o1, gs0, gs1, ws0, ws1):
    wid = lax.axis_index("s") * NUM_CORES + lax.axis_index("c")
    base = wid * PER_W
    gbufs, obufs = (g0, g1), (o0, o1)
    gsems, wsems = (gs0, gs1), (ws0, ws1)

    lanes = lax.iota(jnp.int32, 16)
    cols = [jnp.full((16,), j, jnp.int32) for j in range(OUT)]

    def start_gather(g, p):
        pltpu.async_copy(table_hbm.at[idx_v.at[g]], gbufs[p], gsems[p])

    def wait_gather(g, p):
        pltpu.make_async_copy(table_hbm.at[idx_v.at[g]], gbufs[p],
                              gsems[p]).wait()

    def start_write(g, p):
        pltpu.async_copy(obufs[p], out_hbm.at[pl.ds(base + g * CHUNK, CHUNK)],
                         wsems[p])

    def wait_write(g, p):
        pltpu.make_async_copy(obufs[p],
                              out_hbm.at[pl.ds(base + g * CHUNK, CHUNK)],
                              wsems[p]).wait()

    def normalize(p):
        src_v, dst_v = gbufs[p], obufs[p]

        def group_body(gr, c):
            row_ids = gr * 16 + lanes
            elems = [plsc.load_gather(src_v, [row_ids, cols[j]])
                     for j in range(OUT)]
            acc = jnp.zeros((16,), jnp.float32)
            for e in elems:
                acc = acc + e * e
            inv = _rsqrt(acc)
            for j, e in enumerate(elems):
                plsc.store_scatter(dst_v, [row_ids, cols[j]], e * inv)
            return c

        lax.fori_loop(0, GROUPS, group_body, 0)

    pltpu.sync_copy(idx_hbm.at[wid], idx_v)
    start_gather(0, 0)
    start_gather(1, 1)

    def pair_body(t, carry):
        for p in range(2):
            g = 2 * t + p
            wait_gather(g, p)

            @pl.when(t > 0)
            def _():
                wait_write(g - 2, p)

            normalize(p)
            start_write(g, p)

            @pl.when(g + 2 < N_CHUNKS)
            def _():
                start_gather(g + 2, p)

        return carry

    lax.fori_loop(0, N_CHUNKS // 2, pair_body, 0)
    wait_write(N_CHUNKS - 2, 0)
    wait_write(N_CHUNKS - 1, 1)


def kernel(src, categories_means, categories_logvars):
    del categories_logvars
    idx = src.astype(jnp.int32).reshape(NW, N_CHUNKS, CHUNK)
    flat = _gather_normalize(categories_means, idx)
    return flat.reshape(BATCH, FIELDS, OUT)
